# Initial kernel scaffold; baseline (speedup 1.0000x reference)
import jax
import jax.numpy as jnp
from jax.experimental import pallas as pl


def kernel(input_ids, token_table, pos_table):
    raise NotImplementedError("write your pallas kernel here")


# speedup vs baseline: 1.2619x; 1.2619x over previous
import functools

import jax
import jax.numpy as jnp
from jax import lax
from jax.experimental import pallas as pl
from jax.experimental.pallas import tpu as pltpu
from jax.experimental.pallas import tpu_sc as plsc

VOCAB = 49408
EMBED = 256
BATCH = 4096
SEQ = 77

NUM_CORES = 2
NUM_SUBCORES = 16
NUM_WORKERS = NUM_CORES * NUM_SUBCORES
ROWS = BATCH * SEQ
ROWS_PER_W = ROWS // NUM_WORKERS
CHUNK = 128
NCHUNK = ROWS_PER_W // CHUNK
LANES = 16


def _body(idx_hbm, token_hbm, pos_hbm, out_hbm, idx_v, pos_v, rows_v, sem_g):
    wid = lax.axis_index("s") * NUM_CORES + lax.axis_index("c")

    pltpu.sync_copy(idx_hbm.at[wid], idx_v)
    pltpu.sync_copy(pos_hbm, pos_v)

    def chunk_step(j, p0):
        pltpu.async_copy(token_hbm.at[idx_v.at[j]], rows_v, sem_g).wait()

        def add_row(i, p):
            for c in range(EMBED // LANES):
                sl = pl.ds(c * LANES, LANES)
                rows_v[i, sl] = rows_v[i, sl] + pos_v[p, sl]
            p = p + 1
            return lax.select(p >= SEQ, p - SEQ, p)

        pend = lax.fori_loop(0, CHUNK, add_row, p0)
        start = (wid * NCHUNK + j) * CHUNK
        pltpu.sync_copy(rows_v, out_hbm.at[pl.ds(start, CHUNK)])
        return pend

    lax.fori_loop(0, NCHUNK, chunk_step, 0)


@jax.jit
def _run(idx_blocks, token_table, pos_table):
    mesh = plsc.VectorSubcoreMesh(core_axis_name="c", subcore_axis_name="s")
    f = functools.partial(
        pl.kernel,
        out_type=jax.ShapeDtypeStruct((ROWS, EMBED), jnp.float32),
        mesh=mesh,
        scratch_types=[
            pltpu.VMEM((NCHUNK, CHUNK), jnp.int32),
            pltpu.VMEM((SEQ, EMBED), jnp.float32),
            pltpu.VMEM((CHUNK, EMBED), jnp.float32),
            pltpu.SemaphoreType.DMA,
        ],
    )(_body)
    out = f(idx_blocks, token_table, pos_table)
    return out.reshape(BATCH, SEQ, EMBED)


def kernel(input_ids, token_table, pos_table):
    idx_blocks = input_ids.astype(jnp.int32).reshape(NUM_WORKERS, NCHUNK, CHUNK)
    return _run(idx_blocks, token_table, pos_table)


# speedup vs baseline: 1.4077x; 1.1155x over previous
import functools

import jax
import jax.numpy as jnp
from jax import lax
from jax.experimental import pallas as pl
from jax.experimental.pallas import tpu as pltpu
from jax.experimental.pallas import tpu_sc as plsc

VOCAB = 49408
EMBED = 256
BATCH = 4096
SEQ = 77

NUM_CORES = 2
NUM_SUBCORES = 16
NUM_WORKERS = NUM_CORES * NUM_SUBCORES
ROWS = BATCH * SEQ
ROWS_PER_W = ROWS // NUM_WORKERS
CHUNK = 88
NCHUNK = ROWS_PER_W // CHUNK
NBUF = 4
NGROUP = NCHUNK // NBUF
LANES = 16


def _body(idx_hbm, token_hbm, pos_hbm, out_hbm, idx_v, pos_v,
          b0, b1, b2, b3, g0, g1, g2, g3, s0, s1, s2, s3):
    bufs = (b0, b1, b2, b3)
    gsems = (g0, g1, g2, g3)
    ssems = (s0, s1, s2, s3)

    wid = lax.axis_index("s") * NUM_CORES + lax.axis_index("c")
    chunk0 = wid * NCHUNK

    pltpu.sync_copy(idx_hbm.at[wid], idx_v)
    pltpu.sync_copy(pos_hbm, pos_v)

    def gather_start(j, b):
        pltpu.async_copy(token_hbm.at[idx_v.at[j]], bufs[b], gsems[b])

    def gather_wait(b):
        pltpu.make_async_copy(token_hbm.at[pl.ds(0, CHUNK)], bufs[b], gsems[b]).wait()

    def store_start(j, b):
        start = (chunk0 + j) * CHUNK
        pltpu.async_copy(bufs[b], out_hbm.at[pl.ds(start, CHUNK)], ssems[b])

    def store_wait(b):
        pltpu.make_async_copy(bufs[b], out_hbm.at[pl.ds(0, CHUNK)], ssems[b]).wait()

    def add_pos(b, p):
        buf = bufs[b]

        def add_row(i, p):
            for c in range(EMBED // LANES):
                sl = pl.ds(c * LANES, LANES)
                buf[i, sl] = buf[i, sl] + pos_v[p, sl]
            p = p + 1
            return lax.select(p >= SEQ, p - SEQ, p)

        return lax.fori_loop(0, CHUNK, add_row, p)

    for b in range(NBUF - 1):
        gather_start(b, b)

    p = 0
    for b in range(NBUF):
        j = b
        if j == 0:
            gather_start(NBUF - 1, NBUF - 1)
        else:
            store_wait((b - 1) % NBUF)
            gather_start(j + NBUF - 1, (b - 1) % NBUF)
        gather_wait(b)
        p = add_pos(b, p)
        store_start(j, b)

    def group(g, p):
        j0 = g * NBUF
        for b in range(NBUF):
            j = j0 + b
            store_wait((b - 1) % NBUF)
            gather_start(j + NBUF - 1, (b - 1) % NBUF)
            gather_wait(b)
            p = add_pos(b, p)
            store_start(j, b)
        return p

    p = lax.fori_loop(1, NGROUP - 1, group, p)

    for b in range(NBUF):
        j = (NGROUP - 1) * NBUF + b
        if j + NBUF - 1 < NCHUNK:
            store_wait((b - 1) % NBUF)
            gather_start(j + NBUF - 1, (b - 1) % NBUF)
        gather_wait(b)
        p = add_pos(b, p)
        store_start(j, b)

    for b in range(NBUF):
        store_wait(b)


@jax.jit
def _run(idx_blocks, token_table, pos_table):
    mesh = plsc.VectorSubcoreMesh(core_axis_name="c", subcore_axis_name="s")
    f = functools.partial(
        pl.kernel,
        out_type=jax.ShapeDtypeStruct((ROWS, EMBED), jnp.float32),
        mesh=mesh,
        scratch_types=[
            pltpu.VMEM((NCHUNK, CHUNK), jnp.int32),
            pltpu.VMEM((SEQ, EMBED), jnp.float32),
        ] + [pltpu.VMEM((CHUNK, EMBED), jnp.float32)] * NBUF
          + [pltpu.SemaphoreType.DMA] * (2 * NBUF),
    )(_body)
    out = f(idx_blocks, token_table, pos_table)
    return out.reshape(BATCH, SEQ, EMBED)


def kernel(input_ids, token_table, pos_table):
    idx_blocks = input_ids.astype(jnp.int32).reshape(NUM_WORKERS, NCHUNK, CHUNK)
    return _run(idx_blocks, token_table, pos_table)


# speedup vs baseline: 2.2928x; 1.6288x over previous
import functools

import jax
import jax.numpy as jnp
from jax import lax
from jax.experimental import pallas as pl
from jax.experimental.pallas import tpu as pltpu
from jax.experimental.pallas import tpu_sc as plsc

VOCAB = 49408
EMBED = 256
BATCH = 4096
SEQ = 77

NUM_CORES = 2
NUM_SUBCORES = 16
NUM_WORKERS = NUM_CORES * NUM_SUBCORES
ROWS = BATCH * SEQ
ROWS_PER_W = ROWS // NUM_WORKERS
CHUNK = 88
NCHUNK = ROWS_PER_W // CHUNK
NBUF = 4
NGROUP = NCHUNK // NBUF
LANES = 16


def _body(idx_hbm, token_hbm, pos_hbm, out_hbm, idx_v, pos_v,
          b0, b1, b2, b3, g0, g1, g2, g3, s0, s1, s2, s3):
    bufs = (b0, b1, b2, b3)
    gsems = (g0, g1, g2, g3)
    ssems = (s0, s1, s2, s3)

    wid = lax.axis_index("s") * NUM_CORES + lax.axis_index("c")
    chunk0 = wid * NCHUNK

    pltpu.sync_copy(idx_hbm.at[wid], idx_v)
    pltpu.sync_copy(pos_hbm, pos_v)

    def gather_start(j, b):
        pltpu.async_copy(token_hbm.at[idx_v.at[j]], bufs[b], gsems[b])

    def gather_wait(b):
        pltpu.make_async_copy(token_hbm.at[pl.ds(0, CHUNK)], bufs[b], gsems[b]).wait()

    def store_start(j, b):
        start = (chunk0 + j) * CHUNK
        pltpu.async_copy(bufs[b], out_hbm.at[pl.ds(start, CHUNK)], ssems[b])

    def store_wait(b):
        pltpu.make_async_copy(bufs[b], out_hbm.at[pl.ds(0, CHUNK)], ssems[b]).wait()

    def add_pos(b, p0):
        buf = bufs[b]

        @plsc.parallel_loop(0, CHUNK, 1, unroll=4)
        def _(i):
            p = lax.rem(p0 + i, SEQ)
            for c in range(EMBED // LANES):
                sl = pl.ds(c * LANES, LANES)
                plsc.addupdate(buf.at[i, sl], pos_v[p, sl])

        p0 = p0 + (CHUNK % SEQ)
        return lax.select(p0 >= SEQ, p0 - SEQ, p0)

    for b in range(NBUF - 1):
        gather_start(b, b)

    p = 0
    for b in range(NBUF):
        j = b
        if j == 0:
            gather_start(NBUF - 1, NBUF - 1)
        else:
            store_wait((b - 1) % NBUF)
            gather_start(j + NBUF - 1, (b - 1) % NBUF)
        gather_wait(b)
        p = add_pos(b, p)
        store_start(j, b)

    def group(g, p):
        j0 = g * NBUF
        for b in range(NBUF):
            j = j0 + b
            store_wait((b - 1) % NBUF)
            gather_start(j + NBUF - 1, (b - 1) % NBUF)
            gather_wait(b)
            p = add_pos(b, p)
            store_start(j, b)
        return p

    p = lax.fori_loop(1, NGROUP - 1, group, p)

    for b in range(NBUF):
        j = (NGROUP - 1) * NBUF + b
        if j + NBUF - 1 < NCHUNK:
            store_wait((b - 1) % NBUF)
            gather_start(j + NBUF - 1, (b - 1) % NBUF)
        gather_wait(b)
        p = add_pos(b, p)
        store_start(j, b)

    for b in range(NBUF):
        store_wait(b)


@jax.jit
def _run(idx_blocks, token_table, pos_table):
    mesh = plsc.VectorSubcoreMesh(core_axis_name="c", subcore_axis_name="s")
    f = functools.partial(
        pl.kernel,
        out_type=jax.ShapeDtypeStruct((ROWS, EMBED), jnp.float32),
        mesh=mesh,
        scratch_types=[
            pltpu.VMEM((NCHUNK, CHUNK), jnp.int32),
            pltpu.VMEM((SEQ, EMBED), jnp.float32),
        ] + [pltpu.VMEM((CHUNK, EMBED), jnp.float32)] * NBUF
          + [pltpu.SemaphoreType.DMA] * (2 * NBUF),
    )(_body)
    out = f(idx_blocks, token_table, pos_table)
    return out.reshape(BATCH, SEQ, EMBED)


def kernel(input_ids, token_table, pos_table):
    idx_blocks = input_ids.astype(jnp.int32).reshape(NUM_WORKERS, NCHUNK, CHUNK)
    return _run(idx_blocks, token_table, pos_table)


# speedup vs baseline: 7.5962x; 3.3131x over previous
import functools

import jax
import jax.numpy as jnp
from jax import lax
from jax.experimental import pallas as pl
from jax.experimental.pallas import tpu as pltpu
from jax.experimental.pallas import tpu_sc as plsc

VOCAB = 49408
EMBED = 256
BATCH = 4096
SEQ = 77

NUM_CORES = 2
NUM_SUBCORES = 16
NUM_WORKERS = NUM_CORES * NUM_SUBCORES
BLOCK = BATCH // NUM_WORKERS
NBUF = 3
LANES = 16


def _body(idx_hbm, token_hbm, pos_hbm, out_hbm, idx_v, pos_v,
          b0, b1, b2, g0, g1, g2, s0, s1, s2):
    bufs = (b0, b1, b2)
    gsems = (g0, g1, g2)
    ssems = (s0, s1, s2)

    wid = lax.axis_index("s") * NUM_CORES + lax.axis_index("c")
    base = wid * BLOCK

    pltpu.sync_copy(idx_hbm.at[wid], idx_v)
    pltpu.sync_copy(pos_hbm, pos_v)

    def gather_start(s, b):
        pltpu.async_copy(token_hbm.at[idx_v.at[s]], bufs[b], gsems[b])

    def gather_wait(b):
        pltpu.make_async_copy(token_hbm.at[pl.ds(0, BLOCK)], bufs[b], gsems[b]).wait()

    def store_start(s, b):
        pltpu.async_copy(bufs[b], out_hbm.at[s, pl.ds(base, BLOCK)], ssems[b])

    def store_wait(b):
        pltpu.make_async_copy(bufs[b], out_hbm.at[0, pl.ds(base, BLOCK)], ssems[b]).wait()

    def add_pos(s, b):
        buf = bufs[b]
        regs = [pos_v[s, pl.ds(c * LANES, LANES)] for c in range(EMBED // LANES)]

        @plsc.parallel_loop(0, BLOCK, 1, unroll=4)
        def _(i):
            for c in range(EMBED // LANES):
                plsc.addupdate(buf.at[i, pl.ds(c * LANES, LANES)], regs[c])

    for b in range(NBUF - 1):
        gather_start(b, b)

    for b in range(NBUF):
        s = b
        if s == 0:
            gather_start(NBUF - 1, NBUF - 1)
        else:
            store_wait((b - 1) % NBUF)
            gather_start(s + NBUF - 1, (b - 1) % NBUF)
        gather_wait(b)
        add_pos(s, b)
        store_start(s, b)

    def group(g, carry):
        s0_ = g * NBUF
        for b in range(NBUF):
            s = s0_ + b
            store_wait((b - 1) % NBUF)
            gather_start(s + NBUF - 1, (b - 1) % NBUF)
            gather_wait(b)
            add_pos(s, b)
            store_start(s, b)
        return carry

    lax.fori_loop(1, (SEQ - (NBUF - 1) - NBUF) // NBUF + 1, group, 0)

    for s in range(SEQ - ((SEQ - NBUF) % NBUF), SEQ):
        b = s % NBUF
        store_wait((b - 1) % NBUF)
        gather_wait(b)
        add_pos(s, b)
        store_start(s, b)

    store_wait((SEQ - 1) % NBUF)


@jax.jit
def _run(idx_blocks, token_table, pos_table):
    mesh = plsc.VectorSubcoreMesh(core_axis_name="c", subcore_axis_name="s")
    f = functools.partial(
        pl.kernel,
        out_type=jax.ShapeDtypeStruct((SEQ, BATCH, EMBED), jnp.float32),
        mesh=mesh,
        scratch_types=[
            pltpu.VMEM((SEQ, BLOCK), jnp.int32),
            pltpu.VMEM((SEQ, EMBED), jnp.float32),
        ] + [pltpu.VMEM((BLOCK, EMBED), jnp.float32)] * NBUF
          + [pltpu.SemaphoreType.DMA] * (2 * NBUF),
    )(_body)
    out = f(idx_blocks, token_table, pos_table)
    return out.transpose(1, 0, 2)


def kernel(input_ids, token_table, pos_table):
    idx_blocks = input_ids.astype(jnp.int32).reshape(
        NUM_WORKERS, BLOCK, SEQ).transpose(0, 2, 1)
    return _run(idx_blocks, token_table, pos_table)
